# Initial kernel scaffold; baseline (speedup 1.0000x reference)
import jax
import jax.numpy as jnp
from jax.experimental import pallas as pl


def kernel(x, edge, weight, W1, b1, W3, b3, W7, b7):
    raise NotImplementedError("write your pallas kernel here")


# speedup vs baseline: 1.0008x; 1.0008x over previous
import jax
import jax.numpy as jnp
from jax.experimental import pallas as pl


def _leaky(x):
    return jnp.where(x >= 0, x, 0.01 * x)


def _scatter_mean(vals, idx, num_segments):
    s = jax.ops.segment_sum(vals, idx, num_segments=num_segments)
    c = jax.ops.segment_sum(jnp.ones((idx.shape[0], 1), vals.dtype), idx, num_segments=num_segments)
    return s / jnp.clip(c, 1.0)


def _final_kernel(logits_ref, out_ref):
    l = logits_ref[...]
    m = jnp.max(l, axis=1, keepdims=True)
    e = jnp.exp(l - m)
    out_ref[...] = l - m - jnp.log(jnp.sum(e, axis=1, keepdims=True))


def kernel(x, edge, weight, W1, b1, W3, b3, W7, b7):
    n = x.shape[0]
    row = edge[0]
    col = edge[1]
    out1 = x[col] * weight[:, None]
    out1 = _leaky(out1 @ W1.T + b1)
    out1 = _scatter_mean(out1, row, n)
    out1 = out1 + _leaky(x @ W1.T + b1)
    out2 = out1[col] * weight[:, None]
    out2 = _leaky(out2 @ W3.T + b3)
    out2 = _scatter_mean(out2, row, n)
    out2 = out2 + _leaky(out1 @ W3.T + b3)
    pooled = jnp.mean(out2, axis=0, keepdims=True)
    logits = pooled @ W7.T + b7
    return pl.pallas_call(
        _final_kernel,
        out_shape=jax.ShapeDtypeStruct(logits.shape, logits.dtype),
    )(logits)


# speedup vs baseline: 3.6168x; 3.6139x over previous
import functools

import jax
import jax.numpy as jnp
from jax import lax
from jax.experimental import pallas as pl
from jax.experimental.pallas import tpu as pltpu
from jax.experimental.pallas import tpu_sc as plsc

NC = 2
NS = 16
L = 16
CHUNK = 256
SUB = CHUNK // 128


def _leaky(t):
    return jnp.maximum(t, 0.01 * t)


def _stage_a_body(x_ref, w1_ref, b1_ref, ya_ref, yb_ref, skip_ref):
    y = lax.dot_general(x_ref[...], w1_ref[...], (((1,), (1,)), ((), ())),
                        preferred_element_type=jnp.float32)
    ya_ref[...] = y[:, :16]
    yb_ref[...] = y[:, 16:]
    skip_ref[...] = _leaky(y + b1_ref[...])


def _stage_c_body(s1a_ref, s1b_ref, d0_ref, d1_ref, skip_ref, w3_ref, b3_ref,
                  o0_ref, o1_ref, o2_ref, o3_ref, skip2_ref):
    cnt = d0_ref[:, :1] + d1_ref[:, :1]
    inv = 1.0 / jnp.maximum(cnt, 1.0)
    out1 = jnp.concatenate([s1a_ref[...], s1b_ref[...]], axis=1) * inv + skip_ref[...]
    y2 = lax.dot_general(out1, w3_ref[...], (((1,), (1,)), ((), ())),
                         preferred_element_type=jnp.float32)
    o0_ref[...] = y2[:, 0:16]
    o1_ref[...] = y2[:, 16:32]
    o2_ref[...] = y2[:, 32:48]
    o3_ref[...] = y2[:, 48:64]
    skip2_ref[...] = _leaky(y2 + b3_ref[...])


def _stage_e_body(n_nodes, grid_n, s20_ref, s21_ref, s22_ref, s23_ref,
                  d0_ref, d1_ref, skip2_ref, w7_ref, b7_ref, out_ref, acc_ref):
    i = pl.program_id(0)

    @pl.when(i == 0)
    def _():
        acc_ref[...] = jnp.zeros_like(acc_ref)

    cnt = d0_ref[:, :1] + d1_ref[:, :1]
    inv = 1.0 / jnp.maximum(cnt, 1.0)
    out2 = (jnp.concatenate([s20_ref[...], s21_ref[...], s22_ref[...], s23_ref[...]],
                            axis=1) * inv + skip2_ref[...])
    r = out2.shape[0]
    acc_ref[...] += jnp.sum(out2.reshape(r // 8, 8, 64), axis=0)

    @pl.when(i == grid_n - 1)
    def _():
        pooled = jnp.sum(acc_ref[...], axis=0, keepdims=True) * (1.0 / n_nodes)
        logits = lax.dot_general(pooled, w7_ref[...], (((1,), (1,)), ((), ())),
                                 preferred_element_type=jnp.float32) + b7_ref[...]
        m = jnp.max(logits, axis=1, keepdims=True)
        out_ref[...] = (logits - m) - jnp.log(
            jnp.sum(jnp.exp(logits - m), axis=1, keepdims=True))


def _make_deg_kernel(npad, epad):
    rows_pt = npad // NS
    ept = epad // (NC * NS)
    chunks = ept // CHUNK
    mesh = plsc.VectorSubcoreMesh(core_axis_name="c", subcore_axis_name="s",
                                  num_cores=NC, num_subcores=NS)
    out_type = [jax.ShapeDtypeStruct((npad, 16), jnp.float32) for _ in range(NC)]
    scratch = [
        pltpu.VMEM((SUB, 128), jnp.int32),
        pltpu.VMEM((128, 16), jnp.float32),
        pltpu.MemorySpace.VMEM_SHARED((npad, 16), jnp.float32),
    ]

    @functools.partial(pl.kernel, out_type=out_type, mesh=mesh,
                       scratch_types=scratch,
                       compiler_params=pltpu.CompilerParams(
                           use_tc_tiling_on_sc=False))
    def deg_kernel(row2_hbm, ones_hbm, zeros_hbm, d0_hbm, d1_hbm,
                   rowv, onesv, acc):
        c = lax.axis_index("c")
        s = lax.axis_index("s")
        r0 = s * rows_pt
        pltpu.sync_copy(zeros_hbm.at[pl.ds(r0, rows_pt)],
                        acc.at[pl.ds(r0, rows_pt)])
        pltpu.sync_copy(ones_hbm, onesv)
        plsc.subcore_barrier()
        wid = c * NS + s

        def chunk_body(kk, carry):
            rb = wid * (ept // 128) + kk * SUB
            pltpu.sync_copy(row2_hbm.at[pl.ds(rb, SUB)], rowv)
            for j in range(SUB):
                pltpu.sync_copy(onesv, acc.at[rowv.at[j]], add=True)
            return carry

        lax.fori_loop(0, chunks, chunk_body, 0)
        plsc.subcore_barrier()
        for core in range(NC):
            @pl.when(c == core)
            def _(core=core):
                out = (d0_hbm, d1_hbm)[core]
                pltpu.sync_copy(acc.at[pl.ds(r0, rows_pt)],
                                out.at[pl.ds(r0, rows_pt)])

    return deg_kernel


def _make_pass_kernel(num_groups, npad, epad):
    gpc = num_groups // NC
    rows_pt = npad // NS
    ept = epad // NS
    chunks = ept // CHUNK
    mesh = plsc.VectorSubcoreMesh(core_axis_name="c", subcore_axis_name="s",
                                  num_cores=NC, num_subcores=NS)
    out_type = [jax.ShapeDtypeStruct((npad, 16), jnp.float32)
                for _ in range(num_groups)]
    scratch = [
        pltpu.VMEM((SUB, 128), jnp.int32),
        pltpu.VMEM((SUB, 128), jnp.int32),
        pltpu.VMEM((CHUNK,), jnp.float32),
        pltpu.VMEM((CHUNK, L), jnp.float32),
        pltpu.VMEM((CHUNK, L), jnp.float32),
        pltpu.VMEM((L,), jnp.float32),
        pltpu.MemorySpace.VMEM_SHARED((npad, 16), jnp.float32),
        pltpu.SemaphoreType.DMA,
    ]

    @functools.partial(pl.kernel, out_type=out_type, mesh=mesh,
                       scratch_types=scratch,
                       compiler_params=pltpu.CompilerParams(
                           use_tc_tiling_on_sc=False))
    def pass_kernel(*refs):
        tabs = refs[:num_groups]
        col2_hbm, row2_hbm, wflat_hbm, b_hbm, zeros_hbm = refs[num_groups:num_groups + 5]
        outs = refs[num_groups + 5:2 * num_groups + 5]
        colv, rowv, wv, gath, val, bvec_s, acc, sem = refs[2 * num_groups + 5:]
        c = lax.axis_index("c")
        s = lax.axis_index("s")
        r0 = s * rows_pt

        def run_group(tab, out, gidx):
            pltpu.sync_copy(zeros_hbm.at[pl.ds(r0, rows_pt)],
                            acc.at[pl.ds(r0, rows_pt)])
            pltpu.sync_copy(b_hbm.at[gidx], bvec_s)
            plsc.subcore_barrier()

            def chunk_body(kk, carry):
                rb = s * (ept // 128) + kk * SUB
                pltpu.sync_copy(col2_hbm.at[pl.ds(rb, SUB)], colv)
                pltpu.sync_copy(row2_hbm.at[pl.ds(rb, SUB)], rowv)
                pltpu.sync_copy(wflat_hbm.at[pl.ds(s * ept + kk * CHUNK, CHUNK)],
                                wv)
                cps = [pltpu.async_copy(tab.at[colv.at[j]],
                                        gath.at[pl.ds(j * 128, 128)], sem)
                       for j in range(SUB)]
                for cp in cps:
                    cp.wait()
                bvec = bvec_s[...]

                def grp_body(j, carry2):
                    wg = wv[pl.ds(j * L, L)]
                    for jj in range(L):
                        e = j * L + jj
                        wj = jnp.broadcast_to(
                            lax.slice(wg, (jj,), (jj + 1,)), (L,))
                        t = gath[e, :] * wj + bvec
                        val[e, :] = jnp.maximum(t, 0.01 * t)
                    return carry2

                lax.fori_loop(0, CHUNK // L, grp_body, 0)
                for j in range(SUB):
                    pltpu.sync_copy(val.at[pl.ds(j * 128, 128)],
                                    acc.at[rowv.at[j]], add=True)
                return carry

            lax.fori_loop(0, chunks, chunk_body, 0)
            plsc.subcore_barrier()
            pltpu.sync_copy(acc.at[pl.ds(r0, rows_pt)],
                            out.at[pl.ds(r0, rows_pt)])
            plsc.subcore_barrier()

        for core in range(NC):
            @pl.when(c == core)
            def _(core=core):
                for gl in range(gpc):
                    gidx = core * gpc + gl
                    run_group(tabs[gidx], outs[gidx], gidx)

    return pass_kernel


def kernel(x, edge, weight, W1, b1, W3, b3, W7, b7):
    n, _ = x.shape
    e = edge.shape[1]
    row = edge[0]
    col = edge[1]

    npad = ((n + 1 + 127) // 128) * 128
    estep = NC * NS * CHUNK
    epad = ((e + estep - 1) // estep) * estep
    pad = epad - e
    colp = jnp.concatenate([col, jnp.zeros((pad,), jnp.int32)])
    rowp = jnp.concatenate([row, jnp.full((pad,), n, jnp.int32)])
    wp = jnp.concatenate([weight, jnp.zeros((pad,), jnp.float32)])
    col2 = colp.reshape(-1, 128)
    row2 = rowp.reshape(-1, 128)
    zeros_hbm = jnp.zeros((npad, 16), jnp.float32)
    ones_hbm = jnp.ones((128, 16), jnp.float32)

    blk = 2000
    grid_n = n // blk

    ya, yb, skip1 = pl.pallas_call(
        _stage_a_body,
        grid=(grid_n,),
        in_specs=[
            pl.BlockSpec((blk, x.shape[1]), lambda i: (i, 0)),
            pl.BlockSpec(W1.shape, lambda i: (0, 0)),
            pl.BlockSpec((1, 32), lambda i: (0, 0)),
        ],
        out_specs=[
            pl.BlockSpec((blk, 16), lambda i: (i, 0)),
            pl.BlockSpec((blk, 16), lambda i: (i, 0)),
            pl.BlockSpec((blk, 32), lambda i: (i, 0)),
        ],
        out_shape=[
            jax.ShapeDtypeStruct((n, 16), jnp.float32),
            jax.ShapeDtypeStruct((n, 16), jnp.float32),
            jax.ShapeDtypeStruct((n, 32), jnp.float32),
        ],
    )(x, W1, b1.reshape(1, 32))

    d0, d1 = _make_deg_kernel(npad, epad)(row2, ones_hbm, zeros_hbm)

    s1a, s1b = _make_pass_kernel(2, npad, epad)(
        ya, yb, col2, row2, wp, b1.reshape(2, 16), zeros_hbm)

    nspec = pl.BlockSpec((blk, 16), lambda i: (i, 0))
    y20, y21, y22, y23, skip2 = pl.pallas_call(
        _stage_c_body,
        grid=(grid_n,),
        in_specs=[nspec, nspec, nspec, nspec,
                  pl.BlockSpec((blk, 32), lambda i: (i, 0)),
                  pl.BlockSpec(W3.shape, lambda i: (0, 0)),
                  pl.BlockSpec((1, 64), lambda i: (0, 0))],
        out_specs=[nspec, nspec, nspec, nspec,
                   pl.BlockSpec((blk, 64), lambda i: (i, 0))],
        out_shape=[jax.ShapeDtypeStruct((n, 16), jnp.float32)] * 4
        + [jax.ShapeDtypeStruct((n, 64), jnp.float32)],
    )(s1a, s1b, d0, d1, skip1, W3, b3.reshape(1, 64))

    s20, s21, s22, s23 = _make_pass_kernel(4, npad, epad)(
        y20, y21, y22, y23, col2, row2, wp, b3.reshape(4, 16), zeros_hbm)

    out = pl.pallas_call(
        functools.partial(_stage_e_body, n, grid_n),
        grid=(grid_n,),
        in_specs=[nspec, nspec, nspec, nspec, nspec, nspec,
                  pl.BlockSpec((blk, 64), lambda i: (i, 0)),
                  pl.BlockSpec(W7.shape, lambda i: (0, 0)),
                  pl.BlockSpec((1, 2), lambda i: (0, 0))],
        out_specs=pl.BlockSpec((1, 2), lambda i: (0, 0)),
        out_shape=jax.ShapeDtypeStruct((1, 2), jnp.float32),
        scratch_shapes=[pltpu.VMEM((8, 64), jnp.float32)],
    )(s20, s21, s22, s23, d0, d1, skip2, W7, b7.reshape(1, 2))
    return out


# speedup vs baseline: 7.7634x; 2.1465x over previous
import functools

import jax
import jax.numpy as jnp
from jax import lax
from jax.experimental import pallas as pl
from jax.experimental.pallas import tpu as pltpu
from jax.experimental.pallas import tpu_sc as plsc

NC = 2
NS = 16
L = 16
CHUNK = 256
SUB = CHUNK // 128


def _leaky(t):
    return jnp.maximum(t, 0.01 * t)


def _stage_a_body(x_ref, w1_ref, b1_ref, ya_ref, yb_ref, skip_ref):
    y = lax.dot_general(x_ref[...], w1_ref[...], (((1,), (1,)), ((), ())),
                        preferred_element_type=jnp.float32)
    ya_ref[...] = y[:, :16]
    yb_ref[...] = y[:, 16:]
    skip_ref[...] = _leaky(y + b1_ref[...])


def _stage_c_body(s1a_ref, s1b_ref, d0_ref, d1_ref, skip_ref, w3_ref, b3_ref,
                  o0_ref, o1_ref, o2_ref, o3_ref, skip2_ref):
    cnt = d0_ref[:, :1] + d1_ref[:, :1]
    inv = 1.0 / jnp.maximum(cnt, 1.0)
    out1 = jnp.concatenate([s1a_ref[...], s1b_ref[...]], axis=1) * inv + skip_ref[...]
    y2 = lax.dot_general(out1, w3_ref[...], (((1,), (1,)), ((), ())),
                         preferred_element_type=jnp.float32)
    o0_ref[...] = y2[:, 0:16]
    o1_ref[...] = y2[:, 16:32]
    o2_ref[...] = y2[:, 32:48]
    o3_ref[...] = y2[:, 48:64]
    skip2_ref[...] = _leaky(y2 + b3_ref[...])


def _stage_e_body(n_nodes, grid_n, s20_ref, s21_ref, s22_ref, s23_ref,
                  d0_ref, d1_ref, skip2_ref, w7_ref, b7_ref, out_ref, acc_ref):
    i = pl.program_id(0)

    @pl.when(i == 0)
    def _():
        acc_ref[...] = jnp.zeros_like(acc_ref)

    cnt = d0_ref[:, :1] + d1_ref[:, :1]
    inv = 1.0 / jnp.maximum(cnt, 1.0)
    out2 = (jnp.concatenate([s20_ref[...], s21_ref[...], s22_ref[...], s23_ref[...]],
                            axis=1) * inv + skip2_ref[...])
    r = out2.shape[0]
    acc_ref[...] += jnp.sum(out2.reshape(r // 8, 8, 64), axis=0)

    @pl.when(i == grid_n - 1)
    def _():
        pooled = jnp.sum(acc_ref[...], axis=0, keepdims=True) * (1.0 / n_nodes)
        logits = lax.dot_general(pooled, w7_ref[...], (((1,), (1,)), ((), ())),
                                 preferred_element_type=jnp.float32) + b7_ref[...]
        m = jnp.max(logits, axis=1, keepdims=True)
        out_ref[...] = (logits - m) - jnp.log(
            jnp.sum(jnp.exp(logits - m), axis=1, keepdims=True))


def _make_deg_kernel(npad, epad):
    rows_pt = npad // NS
    ept = epad // (NC * NS)
    chunks = ept // CHUNK
    mesh = plsc.VectorSubcoreMesh(core_axis_name="c", subcore_axis_name="s",
                                  num_cores=NC, num_subcores=NS)
    out_type = [jax.ShapeDtypeStruct((npad, 16), jnp.float32) for _ in range(NC)]
    scratch = [
        pltpu.VMEM((SUB, 128), jnp.int32),
        pltpu.VMEM((128, 16), jnp.float32),
        pltpu.MemorySpace.VMEM_SHARED((npad, 16), jnp.float32),
    ]

    @functools.partial(pl.kernel, out_type=out_type, mesh=mesh,
                       scratch_types=scratch,
                       compiler_params=pltpu.CompilerParams(
                           use_tc_tiling_on_sc=False))
    def deg_kernel(row2_hbm, ones_hbm, zeros_hbm, d0_hbm, d1_hbm,
                   rowv, onesv, acc):
        c = lax.axis_index("c")
        s = lax.axis_index("s")
        r0 = s * rows_pt
        pltpu.sync_copy(zeros_hbm.at[pl.ds(r0, rows_pt)],
                        acc.at[pl.ds(r0, rows_pt)])
        pltpu.sync_copy(ones_hbm, onesv)
        plsc.subcore_barrier()
        wid = c * NS + s

        def chunk_body(kk, carry):
            rb = wid * (ept // 128) + kk * SUB
            pltpu.sync_copy(row2_hbm.at[pl.ds(rb, SUB)], rowv)
            for j in range(SUB):
                pltpu.sync_copy(onesv, acc.at[rowv.at[j]], add=True)
            return carry

        lax.fori_loop(0, chunks, chunk_body, 0)
        plsc.subcore_barrier()
        for core in range(NC):
            @pl.when(c == core)
            def _(core=core):
                out = (d0_hbm, d1_hbm)[core]
                pltpu.sync_copy(acc.at[pl.ds(r0, rows_pt)],
                                out.at[pl.ds(r0, rows_pt)])

    return deg_kernel


def _make_pass_kernel(num_groups, npad, epad):
    gpc = num_groups // NC
    rows_pt = npad // NS
    ept = epad // NS
    chunks = ept // CHUNK
    mesh = plsc.VectorSubcoreMesh(core_axis_name="c", subcore_axis_name="s",
                                  num_cores=NC, num_subcores=NS)
    out_type = [jax.ShapeDtypeStruct((npad, 16), jnp.float32)
                for _ in range(num_groups)]
    scratch = [
        [pltpu.VMEM((SUB, 128), jnp.int32) for _ in range(2)],
        [pltpu.VMEM((SUB, 128), jnp.int32) for _ in range(4)],
        [pltpu.VMEM((CHUNK,), jnp.float32) for _ in range(4)],
        [pltpu.VMEM((CHUNK, L), jnp.float32) for _ in range(2)],
        [pltpu.VMEM((CHUNK, L), jnp.float32) for _ in range(2)],
        pltpu.VMEM((L,), jnp.float32),
        pltpu.MemorySpace.VMEM_SHARED((npad, 16), jnp.float32),
        [pltpu.SemaphoreType.DMA for _ in range(2)],
        [pltpu.SemaphoreType.DMA for _ in range(2)],
        [pltpu.SemaphoreType.DMA for _ in range(2)],
    ]

    assert chunks % 4 == 0
    kout = chunks // 4

    @functools.partial(pl.kernel, out_type=out_type, mesh=mesh,
                       scratch_types=scratch,
                       compiler_params=pltpu.CompilerParams(
                           use_tc_tiling_on_sc=False))
    def pass_kernel(*refs):
        tabs = refs[:num_groups]
        col2_hbm, row2_hbm, wflat_hbm, b_hbm, zeros_hbm = refs[num_groups:num_groups + 5]
        outs = refs[num_groups + 5:2 * num_groups + 5]
        (colv, rowv, wv, gath, val, bvec_s, acc,
         sem_a, sem_g, sem_s) = refs[2 * num_groups + 5:]
        c = lax.axis_index("c")
        s = lax.axis_index("s")
        r0 = s * rows_pt

        def run_group(tab, out, gidx):
            def a_copies(k, u):
                rb = s * (ept // 128) + k * SUB
                eb = s * ept + k * CHUNK
                return [
                    pltpu.make_async_copy(col2_hbm.at[pl.ds(rb, SUB)],
                                          colv[u % 2], sem_a[u % 2]),
                    pltpu.make_async_copy(row2_hbm.at[pl.ds(rb, SUB)],
                                          rowv[u % 4], sem_a[u % 2]),
                    pltpu.make_async_copy(wflat_hbm.at[pl.ds(eb, CHUNK)],
                                          wv[u % 4], sem_a[u % 2]),
                ]

            def g_copies(k, u):
                return [
                    pltpu.make_async_copy(tab.at[colv[u % 2].at[j]],
                                          gath[u % 2].at[pl.ds(j * 128, 128)],
                                          sem_g[u % 2])
                    for j in range(SUB)
                ]

            def s_copies(k, u):
                return [
                    pltpu.make_async_copy(val[u % 2].at[pl.ds(j * 128, 128)],
                                          acc.at[rowv[u % 4].at[j]],
                                          sem_s[u % 2])
                    for j in range(SUB)
                ]

            pltpu.sync_copy(zeros_hbm.at[pl.ds(r0, rows_pt)],
                            acc.at[pl.ds(r0, rows_pt)])
            pltpu.sync_copy(b_hbm.at[gidx], bvec_s)
            plsc.subcore_barrier()

            for cp in a_copies(0, 0):
                cp.start()
            for cp in a_copies(1, 1):
                cp.start()
            for cp in a_copies(0, 0):
                cp.wait()
            for cp in g_copies(0, 0):
                cp.start()

            def outer_body(k0, carry):
                for u in range(4):
                    k = k0 * 4 + u

                    def drain_s(k=k, u=u):
                        for cp in s_copies(k - 2, u + 2):
                            cp.wait()

                    if u < 2:
                        pl.when(k0 > 0)(drain_s)
                    else:
                        drain_s()
                    for cp in g_copies(k, u):
                        cp.wait()

                    def prefetch_a(k=k, u=u):
                        for cp in a_copies(k + 2, u + 2):
                            cp.start()

                    if u < 2:
                        prefetch_a()
                    else:
                        pl.when(k0 < kout - 1)(prefetch_a)

                    def start_g(k=k, u=u):
                        for cp in a_copies(k + 1, u + 1):
                            cp.wait()
                        for cp in g_copies(k + 1, u + 1):
                            cp.start()

                    if u < 3:
                        start_g()
                    else:
                        pl.when(k0 < kout - 1)(start_g)

                    bvec = bvec_s[...]
                    gbuf = gath[u % 2]
                    vbuf = val[u % 2]
                    wbuf = wv[u % 4]

                    def grp_body(j, carry2):
                        wg = wbuf[pl.ds(j * L, L)]
                        for jj in range(L):
                            e = j * L + jj
                            wj = jnp.broadcast_to(
                                lax.slice(wg, (jj,), (jj + 1,)), (L,))
                            t = gbuf[e, :] * wj + bvec
                            vbuf[e, :] = jnp.maximum(t, 0.01 * t)
                        return carry2

                    lax.fori_loop(0, CHUNK // L, grp_body, 0)
                    for cp in s_copies(k, u):
                        cp.start(add=True)
                return carry

            lax.fori_loop(0, kout, outer_body, 0)
            for cp in s_copies(chunks - 2, 2):
                cp.wait()
            for cp in s_copies(chunks - 1, 3):
                cp.wait()
            plsc.subcore_barrier()
            pltpu.sync_copy(acc.at[pl.ds(r0, rows_pt)],
                            out.at[pl.ds(r0, rows_pt)])
            plsc.subcore_barrier()

        for core in range(NC):
            @pl.when(c == core)
            def _(core=core):
                for gl in range(gpc):
                    gidx = core * gpc + gl
                    run_group(tabs[gidx], outs[gidx], gidx)

    return pass_kernel


def kernel(x, edge, weight, W1, b1, W3, b3, W7, b7):
    n, _ = x.shape
    e = edge.shape[1]
    row = edge[0]
    col = edge[1]

    npad = ((n + 1 + 127) // 128) * 128
    estep = NC * NS * CHUNK
    epad = ((e + estep - 1) // estep) * estep
    pad = epad - e
    colp = jnp.concatenate([col, jnp.zeros((pad,), jnp.int32)])
    rowp = jnp.concatenate([row, jnp.full((pad,), n, jnp.int32)])
    wp = jnp.concatenate([weight, jnp.zeros((pad,), jnp.float32)])
    col2 = colp.reshape(-1, 128)
    row2 = rowp.reshape(-1, 128)
    zeros_hbm = jnp.zeros((npad, 16), jnp.float32)
    ones_hbm = jnp.ones((128, 16), jnp.float32)

    blk = 2000
    grid_n = n // blk

    ya, yb, skip1 = pl.pallas_call(
        _stage_a_body,
        grid=(grid_n,),
        in_specs=[
            pl.BlockSpec((blk, x.shape[1]), lambda i: (i, 0)),
            pl.BlockSpec(W1.shape, lambda i: (0, 0)),
            pl.BlockSpec((1, 32), lambda i: (0, 0)),
        ],
        out_specs=[
            pl.BlockSpec((blk, 16), lambda i: (i, 0)),
            pl.BlockSpec((blk, 16), lambda i: (i, 0)),
            pl.BlockSpec((blk, 32), lambda i: (i, 0)),
        ],
        out_shape=[
            jax.ShapeDtypeStruct((n, 16), jnp.float32),
            jax.ShapeDtypeStruct((n, 16), jnp.float32),
            jax.ShapeDtypeStruct((n, 32), jnp.float32),
        ],
    )(x, W1, b1.reshape(1, 32))

    d0, d1 = _make_deg_kernel(npad, epad)(row2, ones_hbm, zeros_hbm)

    s1a, s1b = _make_pass_kernel(2, npad, epad)(
        ya, yb, col2, row2, wp, b1.reshape(2, 16), zeros_hbm)

    nspec = pl.BlockSpec((blk, 16), lambda i: (i, 0))
    y20, y21, y22, y23, skip2 = pl.pallas_call(
        _stage_c_body,
        grid=(grid_n,),
        in_specs=[nspec, nspec, nspec, nspec,
                  pl.BlockSpec((blk, 32), lambda i: (i, 0)),
                  pl.BlockSpec(W3.shape, lambda i: (0, 0)),
                  pl.BlockSpec((1, 64), lambda i: (0, 0))],
        out_specs=[nspec, nspec, nspec, nspec,
                   pl.BlockSpec((blk, 64), lambda i: (i, 0))],
        out_shape=[jax.ShapeDtypeStruct((n, 16), jnp.float32)] * 4
        + [jax.ShapeDtypeStruct((n, 64), jnp.float32)],
    )(s1a, s1b, d0, d1, skip1, W3, b3.reshape(1, 64))

    s20, s21, s22, s23 = _make_pass_kernel(4, npad, epad)(
        y20, y21, y22, y23, col2, row2, wp, b3.reshape(4, 16), zeros_hbm)

    out = pl.pallas_call(
        functools.partial(_stage_e_body, n, grid_n),
        grid=(grid_n,),
        in_specs=[nspec, nspec, nspec, nspec, nspec, nspec,
                  pl.BlockSpec((blk, 64), lambda i: (i, 0)),
                  pl.BlockSpec(W7.shape, lambda i: (0, 0)),
                  pl.BlockSpec((1, 2), lambda i: (0, 0))],
        out_specs=pl.BlockSpec((1, 2), lambda i: (0, 0)),
        out_shape=jax.ShapeDtypeStruct((1, 2), jnp.float32),
        scratch_shapes=[pltpu.VMEM((8, 64), jnp.float32)],
    )(s20, s21, s22, s23, d0, d1, skip2, W7, b7.reshape(1, 2))
    return out


# speedup vs baseline: 9.6247x; 1.2397x over previous
import functools

import jax
import jax.numpy as jnp
from jax import lax
from jax.experimental import pallas as pl
from jax.experimental.pallas import tpu as pltpu
from jax.experimental.pallas import tpu_sc as plsc

NC = 2
NS = 16
L = 16
CHUNK = 256
SUB = CHUNK // 128


def _leaky(t):
    return jnp.maximum(t, 0.01 * t)


def _stage_a_body(x_ref, w1_ref, b1_ref, ya_ref, yb_ref, skip_ref):
    y = lax.dot_general(x_ref[...], w1_ref[...], (((1,), (1,)), ((), ())),
                        preferred_element_type=jnp.float32)
    ya_ref[...] = y[:, :16]
    yb_ref[...] = y[:, 16:]
    skip_ref[...] = _leaky(y + b1_ref[...])


def _stage_c_body(s1a_ref, s1b_ref, d0_ref, d1_ref, skip_ref, w3_ref, b3_ref,
                  o0_ref, o1_ref, o2_ref, o3_ref, skip2_ref):
    cnt = d0_ref[:, :1] + d1_ref[:, :1]
    inv = 1.0 / jnp.maximum(cnt, 1.0)
    out1 = jnp.concatenate([s1a_ref[...], s1b_ref[...]], axis=1) * inv + skip_ref[...]
    y2 = lax.dot_general(out1, w3_ref[...], (((1,), (1,)), ((), ())),
                         preferred_element_type=jnp.float32)
    o0_ref[...] = y2[:, 0:16]
    o1_ref[...] = y2[:, 16:32]
    o2_ref[...] = y2[:, 32:48]
    o3_ref[...] = y2[:, 48:64]
    skip2_ref[...] = _leaky(y2 + b3_ref[...])


def _stage_e_body(n_nodes, grid_n, s20_ref, s21_ref, s22_ref, s23_ref,
                  d0_ref, d1_ref, skip2_ref, w7_ref, b7_ref, out_ref, acc_ref):
    i = pl.program_id(0)

    @pl.when(i == 0)
    def _():
        acc_ref[...] = jnp.zeros_like(acc_ref)

    cnt = d0_ref[:, :1] + d1_ref[:, :1]
    inv = 1.0 / jnp.maximum(cnt, 1.0)
    out2 = (jnp.concatenate([s20_ref[...], s21_ref[...], s22_ref[...], s23_ref[...]],
                            axis=1) * inv + skip2_ref[...])
    r = out2.shape[0]
    acc_ref[...] += jnp.sum(out2.reshape(r // 8, 8, 64), axis=0)

    @pl.when(i == grid_n - 1)
    def _():
        pooled = jnp.sum(acc_ref[...], axis=0, keepdims=True) * (1.0 / n_nodes)
        logits = lax.dot_general(pooled, w7_ref[...], (((1,), (1,)), ((), ())),
                                 preferred_element_type=jnp.float32) + b7_ref[...]
        m = jnp.max(logits, axis=1, keepdims=True)
        out_ref[...] = (logits - m) - jnp.log(
            jnp.sum(jnp.exp(logits - m), axis=1, keepdims=True))


def _make_pass_kernel(num_groups, npad, epad, with_deg):
    gpc = num_groups // NC
    rows_pt = npad // NS
    ept = epad // NS
    chunks = ept // CHUNK
    ept2 = epad // (NC * NS)
    chunks2 = ept2 // CHUNK
    mesh = plsc.VectorSubcoreMesh(core_axis_name="c", subcore_axis_name="s",
                                  num_cores=NC, num_subcores=NS)
    nout = num_groups + (2 if with_deg else 0)
    out_type = [jax.ShapeDtypeStruct((npad, 16), jnp.float32)
                for _ in range(nout)]
    scratch = [
        [pltpu.VMEM((SUB, 128), jnp.int32) for _ in range(4)],
        [pltpu.VMEM((SUB, 128), jnp.int32) for _ in range(4)],
        [pltpu.VMEM((CHUNK,), jnp.float32) for _ in range(4)],
        [pltpu.VMEM((CHUNK, L), jnp.float32) for _ in range(4)],
        [pltpu.VMEM((CHUNK, L), jnp.float32) for _ in range(2)],
        pltpu.VMEM((L,), jnp.float32),
        pltpu.MemorySpace.VMEM_SHARED((npad, 16), jnp.float32),
        [pltpu.SemaphoreType.DMA for _ in range(4)],
        [pltpu.SemaphoreType.DMA for _ in range(4)],
        [pltpu.SemaphoreType.DMA for _ in range(4)],
        [pltpu.SemaphoreType.DMA for _ in range(4)],
        [pltpu.SemaphoreType.DMA for _ in range(2)],
    ]

    assert chunks % 4 == 0 and chunks2 % 4 == 0
    kout = chunks // 4
    kout2 = chunks2 // 4

    @functools.partial(pl.kernel, out_type=out_type, mesh=mesh,
                       scratch_types=scratch,
                       compiler_params=pltpu.CompilerParams(
                           use_tc_tiling_on_sc=False))
    def pass_kernel(*refs):
        tabs = refs[:num_groups]
        col2_hbm, row2_hbm, wflat_hbm, b_hbm, zeros_hbm = refs[num_groups:num_groups + 5]
        if with_deg:
            ones_hbm = refs[num_groups + 5]
            outs = refs[num_groups + 6:2 * num_groups + 6]
            d_hbm = refs[2 * num_groups + 6:2 * num_groups + 8]
            rest = refs[2 * num_groups + 8:]
        else:
            outs = refs[num_groups + 5:2 * num_groups + 5]
            d_hbm = None
            rest = refs[2 * num_groups + 5:]
        (colv, rowv, wv, gath, val, bvec_s, acc,
         sem_ac, sem_ar, sem_aw, sem_g, sem_s) = rest
        c = lax.axis_index("c")
        s = lax.axis_index("s")
        r0 = s * rows_pt

        def zero_acc():
            pltpu.sync_copy(zeros_hbm.at[pl.ds(r0, rows_pt)],
                            acc.at[pl.ds(r0, rows_pt)])

        def flush_acc(out):
            pltpu.sync_copy(acc.at[pl.ds(r0, rows_pt)],
                            out.at[pl.ds(r0, rows_pt)])

        if with_deg:
            wid = c * NS + s

            def ar2(k, u):
                rb = wid * (ept2 // 128) + k * SUB
                return pltpu.make_async_copy(row2_hbm.at[pl.ds(rb, SUB)],
                                             rowv[u % 4], sem_ar[u % 4])

            def s2_copies(k, u):
                return [
                    pltpu.make_async_copy(val[1].at[pl.ds(0, 128)],
                                          acc.at[rowv[u % 4].at[j]],
                                          sem_s[u % 2])
                    for j in range(SUB)
                ]

            zero_acc()
            pltpu.sync_copy(ones_hbm, val[1].at[pl.ds(0, 128)])
            plsc.subcore_barrier()
            ar2(0, 0).start()
            ar2(1, 1).start()

            def deg_body(k0, carry):
                for u in range(4):
                    k = k0 * 4 + u

                    def drain(k=k, u=u):
                        for cp in s2_copies(k - 2, u + 2):
                            cp.wait()

                    if u < 2:
                        pl.when(k0 > 0)(drain)
                    else:
                        drain()

                    def prefetch(k=k, u=u):
                        ar2(k + 2, u + 2).start()

                    if u < 2:
                        prefetch()
                    else:
                        pl.when(k0 < kout2 - 1)(prefetch)
                    ar2(k, u).wait()
                    for cp in s2_copies(k, u):
                        cp.start(add=True)
                return carry

            lax.fori_loop(0, kout2, deg_body, 0)
            for cp in s2_copies(chunks2 - 2, 2):
                cp.wait()
            for cp in s2_copies(chunks2 - 1, 3):
                cp.wait()
            plsc.subcore_barrier()
            for core in range(NC):
                @pl.when(c == core)
                def _(core=core):
                    flush_acc(d_hbm[core])
            plsc.subcore_barrier()

        def run_group(tab, out, gidx):
            def ac(k, u):
                rb = s * (ept // 128) + k * SUB
                return pltpu.make_async_copy(col2_hbm.at[pl.ds(rb, SUB)],
                                             colv[u % 4], sem_ac[u % 4])

            def ar(k, u):
                rb = s * (ept // 128) + k * SUB
                return pltpu.make_async_copy(row2_hbm.at[pl.ds(rb, SUB)],
                                             rowv[u % 4], sem_ar[u % 4])

            def aw(k, u):
                eb = s * ept + k * CHUNK
                return pltpu.make_async_copy(wflat_hbm.at[pl.ds(eb, CHUNK)],
                                             wv[u % 4], sem_aw[u % 4])

            def g_copies(k, u):
                return [
                    pltpu.make_async_copy(tab.at[colv[u % 4].at[j]],
                                          gath[u % 4].at[pl.ds(j * 128, 128)],
                                          sem_g[u % 4])
                    for j in range(SUB)
                ]

            def s_copies(k, u):
                return [
                    pltpu.make_async_copy(val[u % 2].at[pl.ds(j * 128, 128)],
                                          acc.at[rowv[u % 4].at[j]],
                                          sem_s[u % 2])
                    for j in range(SUB)
                ]

            zero_acc()
            pltpu.sync_copy(b_hbm.at[gidx], bvec_s)
            plsc.subcore_barrier()

            for j in range(4):
                ac(j, j).start()
            for j in range(2):
                ar(j, j).start()
                aw(j, j).start()
            ac(0, 0).wait()
            for cp in g_copies(0, 0):
                cp.start()
            ac(1, 1).wait()
            for cp in g_copies(1, 1):
                cp.start()

            def outer_body(k0, carry):
                for u in range(4):
                    k = k0 * 4 + u

                    def drain_s(k=k, u=u):
                        for cp in s_copies(k - 2, u + 2):
                            cp.wait()

                    if u < 2:
                        pl.when(k0 > 0)(drain_s)
                    else:
                        drain_s()

                    def prefetch_rw(k=k, u=u):
                        ar(k + 2, u + 2).start()
                        aw(k + 2, u + 2).start()

                    if u < 2:
                        prefetch_rw()
                    else:
                        pl.when(k0 < kout - 1)(prefetch_rw)

                    for cp in g_copies(k, u):
                        cp.wait()

                    def prefetch_c(k=k, u=u):
                        ac(k + 4, u).start()

                    pl.when(k0 < kout - 1)(prefetch_c)

                    def start_g(k=k, u=u):
                        ac(k + 2, u + 2).wait()
                        for cp in g_copies(k + 2, u + 2):
                            cp.start()

                    if u < 2:
                        start_g()
                    else:
                        pl.when(k0 < kout - 1)(start_g)

                    aw(k, u).wait()
                    bvec = bvec_s[...]
                    gbuf = gath[u % 4]
                    vbuf = val[u % 2]
                    wbuf = wv[u % 4]

                    def grp_body(j, carry2):
                        wg = wbuf[pl.ds(j * L, L)]
                        for jj in range(L):
                            e = j * L + jj
                            wj = jnp.broadcast_to(
                                lax.slice(wg, (jj,), (jj + 1,)), (L,))
                            t = gbuf[e, :] * wj + bvec
                            vbuf[e, :] = jnp.maximum(t, 0.01 * t)
                        return carry2

                    lax.fori_loop(0, CHUNK // L, grp_body, 0)
                    ar(k, u).wait()
                    for cp in s_copies(k, u):
                        cp.start(add=True)
                return carry

            lax.fori_loop(0, kout, outer_body, 0)
            for cp in s_copies(chunks - 2, 2):
                cp.wait()
            for cp in s_copies(chunks - 1, 3):
                cp.wait()
            plsc.subcore_barrier()
            flush_acc(out)
            plsc.subcore_barrier()

        for core in range(NC):
            @pl.when(c == core)
            def _(core=core):
                for gl in range(gpc):
                    gidx = core * gpc + gl
                    run_group(tabs[gidx], outs[gidx], gidx)

    return pass_kernel


def kernel(x, edge, weight, W1, b1, W3, b3, W7, b7):
    n, _ = x.shape
    e = edge.shape[1]
    row = edge[0]
    col = edge[1]

    npad = ((n + 1 + 127) // 128) * 128
    estep = NC * NS * CHUNK
    epad = ((e + estep - 1) // estep) * estep
    pad = epad - e
    colp = jnp.concatenate([col, jnp.zeros((pad,), jnp.int32)])
    rowp = jnp.concatenate([row, jnp.full((pad,), n, jnp.int32)])
    wp = jnp.concatenate([weight, jnp.zeros((pad,), jnp.float32)])
    col2 = colp.reshape(-1, 128)
    row2 = rowp.reshape(-1, 128)
    zeros_hbm = jnp.zeros((npad, 16), jnp.float32)
    ones_hbm = jnp.ones((128, 16), jnp.float32)

    blk = 2000
    grid_n = n // blk

    ya, yb, skip1 = pl.pallas_call(
        _stage_a_body,
        grid=(grid_n,),
        in_specs=[
            pl.BlockSpec((blk, x.shape[1]), lambda i: (i, 0)),
            pl.BlockSpec(W1.shape, lambda i: (0, 0)),
            pl.BlockSpec((1, 32), lambda i: (0, 0)),
        ],
        out_specs=[
            pl.BlockSpec((blk, 16), lambda i: (i, 0)),
            pl.BlockSpec((blk, 16), lambda i: (i, 0)),
            pl.BlockSpec((blk, 32), lambda i: (i, 0)),
        ],
        out_shape=[
            jax.ShapeDtypeStruct((n, 16), jnp.float32),
            jax.ShapeDtypeStruct((n, 16), jnp.float32),
            jax.ShapeDtypeStruct((n, 32), jnp.float32),
        ],
    )(x, W1, b1.reshape(1, 32))

    s1a, s1b, d0, d1 = _make_pass_kernel(2, npad, epad, True)(
        ya, yb, col2, row2, wp, b1.reshape(2, 16), zeros_hbm, ones_hbm)

    nspec = pl.BlockSpec((blk, 16), lambda i: (i, 0))
    y20, y21, y22, y23, skip2 = pl.pallas_call(
        _stage_c_body,
        grid=(grid_n,),
        in_specs=[nspec, nspec, nspec, nspec,
                  pl.BlockSpec((blk, 32), lambda i: (i, 0)),
                  pl.BlockSpec(W3.shape, lambda i: (0, 0)),
                  pl.BlockSpec((1, 64), lambda i: (0, 0))],
        out_specs=[nspec, nspec, nspec, nspec,
                   pl.BlockSpec((blk, 64), lambda i: (i, 0))],
        out_shape=[jax.ShapeDtypeStruct((n, 16), jnp.float32)] * 4
        + [jax.ShapeDtypeStruct((n, 64), jnp.float32)],
    )(s1a, s1b, d0, d1, skip1, W3, b3.reshape(1, 64))

    s20, s21, s22, s23 = _make_pass_kernel(4, npad, epad, False)(
        y20, y21, y22, y23, col2, row2, wp, b3.reshape(4, 16), zeros_hbm)

    out = pl.pallas_call(
        functools.partial(_stage_e_body, n, grid_n),
        grid=(grid_n,),
        in_specs=[nspec, nspec, nspec, nspec, nspec, nspec,
                  pl.BlockSpec((blk, 64), lambda i: (i, 0)),
                  pl.BlockSpec(W7.shape, lambda i: (0, 0)),
                  pl.BlockSpec((1, 2), lambda i: (0, 0))],
        out_specs=pl.BlockSpec((1, 2), lambda i: (0, 0)),
        out_shape=jax.ShapeDtypeStruct((1, 2), jnp.float32),
        scratch_shapes=[pltpu.VMEM((8, 64), jnp.float32)],
    )(s20, s21, s22, s23, d0, d1, skip2, W7, b7.reshape(1, 2))
    return out


# speedup vs baseline: 10.5097x; 1.0919x over previous
import functools

import jax
import jax.numpy as jnp
from jax import lax
from jax.experimental import pallas as pl
from jax.experimental.pallas import tpu as pltpu
from jax.experimental.pallas import tpu_sc as plsc

NC = 2
NS = 16
L = 16
CHUNK = 256
SUB = CHUNK // 128


def _leaky(t):
    return jnp.maximum(t, 0.01 * t)


def _stage_a_body(x_ref, w1_ref, b1_ref, ya_ref, yb_ref, skip_ref):
    y = lax.dot_general(x_ref[...], w1_ref[...], (((1,), (1,)), ((), ())),
                        preferred_element_type=jnp.float32)
    ya_ref[...] = y[:, :16]
    yb_ref[...] = y[:, 16:]
    skip_ref[...] = _leaky(y + b1_ref[...])


def _stage_c_body(s1a_ref, s1b_ref, d0_ref, d1_ref, skip_ref, w3_ref,
                  b3_ref, o0_ref, o1_ref, o2_ref, o3_ref, skip2_ref):
    cnt = d0_ref[:, :1] + d1_ref[:, :1]
    inv = 1.0 / jnp.maximum(cnt, 1.0)
    out1 = (jnp.concatenate([s1a_ref[:, :16], s1b_ref[:, :16]], axis=1)
            * inv + skip_ref[...])
    y2 = lax.dot_general(out1, w3_ref[...], (((1,), (1,)), ((), ())),
                         preferred_element_type=jnp.float32)
    o0_ref[...] = y2[:, 0:16]
    o1_ref[...] = y2[:, 16:32]
    o2_ref[...] = y2[:, 32:48]
    o3_ref[...] = y2[:, 48:64]
    skip2_ref[...] = _leaky(y2 + b3_ref[...])


def _stage_e_body(n_nodes, grid_n, s20_ref, s21_ref, s22_ref, s23_ref,
                  d0_ref, d1_ref, skip2_ref, w7_ref, b7_ref, out_ref, acc_ref):
    i = pl.program_id(0)

    @pl.when(i == 0)
    def _():
        acc_ref[...] = jnp.zeros_like(acc_ref)

    cnt = d0_ref[:, :1] + d1_ref[:, :1]
    inv = 1.0 / jnp.maximum(cnt, 1.0)
    out2 = (jnp.concatenate([s20_ref[:, :16], s21_ref[:, :16],
                             s22_ref[:, :16], s23_ref[:, :16]], axis=1)
            * inv + skip2_ref[...])
    r = out2.shape[0]
    acc_ref[...] += jnp.sum(out2.reshape(r // 8, 8, 64), axis=0)

    @pl.when(i == grid_n - 1)
    def _():
        pooled = jnp.sum(acc_ref[...], axis=0, keepdims=True) * (1.0 / n_nodes)
        logits = lax.dot_general(pooled, w7_ref[...], (((1,), (1,)), ((), ())),
                                 preferred_element_type=jnp.float32) + b7_ref[...]
        m = jnp.max(logits, axis=1, keepdims=True)
        out_ref[...] = (logits - m) - jnp.log(
            jnp.sum(jnp.exp(logits - m), axis=1, keepdims=True))


def _make_pass_kernel(num_groups, npad, epad, with_deg):
    gpc = num_groups // NC
    rows_pt = npad // NS
    ept = epad // NS
    chunks = ept // CHUNK
    ept2 = epad // (NC * NS)
    chunks2 = ept2 // CHUNK
    mesh = plsc.VectorSubcoreMesh(core_axis_name="c", subcore_axis_name="s",
                                  num_cores=NC, num_subcores=NS)
    nout = num_groups + (2 if with_deg else 0)
    out_type = [jax.ShapeDtypeStruct((npad, 128), jnp.float32)
                for _ in range(nout)]
    scratch = [
        [pltpu.VMEM((SUB, 128), jnp.int32) for _ in range(4)],
        [pltpu.VMEM((SUB, 128), jnp.int32) for _ in range(4)],
        [pltpu.VMEM((CHUNK,), jnp.float32) for _ in range(4)],
        [pltpu.VMEM((CHUNK, L), jnp.float32) for _ in range(4)],
        [pltpu.VMEM((CHUNK, L), jnp.float32) for _ in range(2)],
        pltpu.VMEM((L,), jnp.float32),
        pltpu.MemorySpace.VMEM_SHARED((npad, 16), jnp.float32),
        [pltpu.SemaphoreType.DMA for _ in range(4)],
        [pltpu.SemaphoreType.DMA for _ in range(4)],
        [pltpu.SemaphoreType.DMA for _ in range(4)],
        [pltpu.SemaphoreType.DMA for _ in range(4)],
        [pltpu.SemaphoreType.DMA for _ in range(2)],
    ]

    assert chunks % 4 == 0 and chunks2 % 4 == 0
    kout = chunks // 4
    kout2 = chunks2 // 4

    @functools.partial(pl.kernel, out_type=out_type, mesh=mesh,
                       scratch_types=scratch,
                       compiler_params=pltpu.CompilerParams(
                           use_tc_tiling_on_sc=False))
    def pass_kernel(*refs):
        tabs = refs[:num_groups]
        col2_hbm, row2_hbm, wflat_hbm, b_hbm, zeros_hbm = refs[num_groups:num_groups + 5]
        if with_deg:
            ones_hbm = refs[num_groups + 5]
            outs = refs[num_groups + 6:2 * num_groups + 6]
            d_hbm = refs[2 * num_groups + 6:2 * num_groups + 8]
            rest = refs[2 * num_groups + 8:]
        else:
            outs = refs[num_groups + 5:2 * num_groups + 5]
            d_hbm = None
            rest = refs[2 * num_groups + 5:]
        (colv, rowv, wv, gath, val, bvec_s, acc,
         sem_ac, sem_ar, sem_aw, sem_g, sem_s) = rest
        c = lax.axis_index("c")
        s = lax.axis_index("s")
        r0 = s * rows_pt

        def zero_acc():
            pltpu.sync_copy(zeros_hbm.at[pl.ds(r0, rows_pt)],
                            acc.at[pl.ds(r0, rows_pt)])

        def flush_acc(out):
            pltpu.sync_copy(acc.at[pl.ds(r0, rows_pt)],
                            out.at[pl.ds(r0, rows_pt), pl.ds(0, 16)])

        if with_deg:
            wid = c * NS + s

            def ar2(k, u):
                rb = wid * (ept2 // 128) + k * SUB
                return pltpu.make_async_copy(row2_hbm.at[pl.ds(rb, SUB)],
                                             rowv[u % 4], sem_ar[u % 4])

            def s2_copies(k, u):
                return [
                    pltpu.make_async_copy(val[1].at[pl.ds(0, 128)],
                                          acc.at[rowv[u % 4].at[j]],
                                          sem_s[u % 2])
                    for j in range(SUB)
                ]

            zero_acc()
            pltpu.sync_copy(ones_hbm, val[1].at[pl.ds(0, 128)])
            plsc.subcore_barrier()
            ar2(0, 0).start()
            ar2(1, 1).start()

            def deg_body(k0, carry):
                for u in range(4):
                    k = k0 * 4 + u

                    def drain(k=k, u=u):
                        for cp in s2_copies(k - 2, u + 2):
                            cp.wait()

                    if u < 2:
                        pl.when(k0 > 0)(drain)
                    else:
                        drain()

                    def prefetch(k=k, u=u):
                        ar2(k + 2, u + 2).start()

                    if u < 2:
                        prefetch()
                    else:
                        pl.when(k0 < kout2 - 1)(prefetch)
                    ar2(k, u).wait()
                    for cp in s2_copies(k, u):
                        cp.start(add=True)
                return carry

            lax.fori_loop(0, kout2, deg_body, 0)
            for cp in s2_copies(chunks2 - 2, 2):
                cp.wait()
            for cp in s2_copies(chunks2 - 1, 3):
                cp.wait()
            plsc.subcore_barrier()
            for core in range(NC):
                @pl.when(c == core)
                def _(core=core):
                    flush_acc(d_hbm[core])
            plsc.subcore_barrier()

        def run_group(tab, out, gidx):
            def ac(k, u):
                rb = s * (ept // 128) + k * SUB
                return pltpu.make_async_copy(col2_hbm.at[pl.ds(rb, SUB)],
                                             colv[u % 4], sem_ac[u % 4])

            def ar(k, u):
                rb = s * (ept // 128) + k * SUB
                return pltpu.make_async_copy(row2_hbm.at[pl.ds(rb, SUB)],
                                             rowv[u % 4], sem_ar[u % 4])

            def aw(k, u):
                eb = s * ept + k * CHUNK
                return pltpu.make_async_copy(wflat_hbm.at[pl.ds(eb, CHUNK)],
                                             wv[u % 4], sem_aw[u % 4])

            def g_copies(k, u):
                return [
                    pltpu.make_async_copy(tab.at[colv[u % 4].at[j]],
                                          gath[u % 4].at[pl.ds(j * 128, 128)],
                                          sem_g[u % 4])
                    for j in range(SUB)
                ]

            def s_copies(k, u):
                return [
                    pltpu.make_async_copy(val[u % 2].at[pl.ds(j * 128, 128)],
                                          acc.at[rowv[u % 4].at[j]],
                                          sem_s[u % 2])
                    for j in range(SUB)
                ]

            zero_acc()
            pltpu.sync_copy(b_hbm.at[gidx], bvec_s)
            plsc.subcore_barrier()

            for j in range(4):
                ac(j, j).start()
            for j in range(2):
                ar(j, j).start()
                aw(j, j).start()
            ac(0, 0).wait()
            for cp in g_copies(0, 0):
                cp.start()
            ac(1, 1).wait()
            for cp in g_copies(1, 1):
                cp.start()

            def outer_body(k0, carry):
                for u in range(4):
                    k = k0 * 4 + u

                    def drain_s(k=k, u=u):
                        for cp in s_copies(k - 2, u + 2):
                            cp.wait()

                    if u < 2:
                        pl.when(k0 > 0)(drain_s)
                    else:
                        drain_s()

                    def prefetch_rw(k=k, u=u):
                        ar(k + 2, u + 2).start()
                        aw(k + 2, u + 2).start()

                    if u < 2:
                        prefetch_rw()
                    else:
                        pl.when(k0 < kout - 1)(prefetch_rw)

                    for cp in g_copies(k, u):
                        cp.wait()

                    def prefetch_c(k=k, u=u):
                        ac(k + 4, u).start()

                    pl.when(k0 < kout - 1)(prefetch_c)

                    def start_g(k=k, u=u):
                        ac(k + 2, u + 2).wait()
                        for cp in g_copies(k + 2, u + 2):
                            cp.start()

                    if u < 2:
                        start_g()
                    else:
                        pl.when(k0 < kout - 1)(start_g)

                    aw(k, u).wait()
                    bvec = bvec_s[...]
                    gbuf = gath[u % 4]
                    vbuf = val[u % 2]
                    wbuf = wv[u % 4]

                    def grp_body(j, carry2):
                        wg = wbuf[pl.ds(j * L, L)]
                        for jj in range(L):
                            e = j * L + jj
                            wj = jnp.broadcast_to(
                                lax.slice(wg, (jj,), (jj + 1,)), (L,))
                            t = gbuf[e, :] * wj + bvec
                            vbuf[e, :] = jnp.maximum(t, 0.01 * t)
                        return carry2

                    lax.fori_loop(0, CHUNK // L, grp_body, 0)
                    ar(k, u).wait()
                    for cp in s_copies(k, u):
                        cp.start(add=True)
                return carry

            lax.fori_loop(0, kout, outer_body, 0)
            for cp in s_copies(chunks - 2, 2):
                cp.wait()
            for cp in s_copies(chunks - 1, 3):
                cp.wait()
            plsc.subcore_barrier()
            flush_acc(out)
            plsc.subcore_barrier()

        for core in range(NC):
            @pl.when(c == core)
            def _(core=core):
                for gl in range(gpc):
                    gidx = core * gpc + gl
                    run_group(tabs[gidx], outs[gidx], gidx)

    return pass_kernel


def kernel(x, edge, weight, W1, b1, W3, b3, W7, b7):
    n, _ = x.shape
    e = edge.shape[1]
    row = edge[0]
    col = edge[1]

    npad = ((n + 1 + 127) // 128) * 128
    estep = NC * NS * CHUNK
    epad = ((e + estep - 1) // estep) * estep
    pad = epad - e
    colp = jnp.concatenate([col, jnp.zeros((pad,), jnp.int32)])
    rowp = jnp.concatenate([row, jnp.full((pad,), n, jnp.int32)])
    wp = jnp.concatenate([weight, jnp.zeros((pad,), jnp.float32)])
    col2 = colp.reshape(-1, 128)
    row2 = rowp.reshape(-1, 128)
    zeros_hbm = jnp.zeros((npad, 16), jnp.float32)
    ones_hbm = jnp.ones((128, 16), jnp.float32)

    blk = 2000
    grid_n = n // blk
    wspec = pl.BlockSpec((blk, 128), lambda i: (i, 0))

    ya, yb, skip1 = pl.pallas_call(
        _stage_a_body,
        grid=(grid_n,),
        in_specs=[
            pl.BlockSpec((blk, x.shape[1]), lambda i: (i, 0)),
            pl.BlockSpec(W1.shape, lambda i: (0, 0)),
            pl.BlockSpec((1, 32), lambda i: (0, 0)),
        ],
        out_specs=[
            pl.BlockSpec((blk, 16), lambda i: (i, 0)),
            pl.BlockSpec((blk, 16), lambda i: (i, 0)),
            pl.BlockSpec((blk, 32), lambda i: (i, 0)),
        ],
        out_shape=[
            jax.ShapeDtypeStruct((n, 16), jnp.float32),
            jax.ShapeDtypeStruct((n, 16), jnp.float32),
            jax.ShapeDtypeStruct((n, 32), jnp.float32),
        ],
    )(x, W1, b1.reshape(1, 32))

    s1a, s1b, d0, d1 = _make_pass_kernel(2, npad, epad, True)(
        ya, yb, col2, row2, wp, b1.reshape(2, 16), zeros_hbm, ones_hbm)

    y20, y21, y22, y23, skip2 = pl.pallas_call(
        _stage_c_body,
        grid=(grid_n,),
        in_specs=[wspec, wspec, wspec, wspec,
                  pl.BlockSpec((blk, 32), lambda i: (i, 0)),
                  pl.BlockSpec(W3.shape, lambda i: (0, 0)),
                  pl.BlockSpec((1, 64), lambda i: (0, 0))],
        out_specs=[pl.BlockSpec((blk, 16), lambda i: (i, 0))] * 4
        + [pl.BlockSpec((blk, 64), lambda i: (i, 0))],
        out_shape=[jax.ShapeDtypeStruct((n, 16), jnp.float32)] * 4
        + [jax.ShapeDtypeStruct((n, 64), jnp.float32)],
    )(s1a, s1b, d0, d1, skip1, W3, b3.reshape(1, 64))

    s20, s21, s22, s23 = _make_pass_kernel(4, npad, epad, False)(
        y20, y21, y22, y23, col2, row2, wp, b3.reshape(4, 16), zeros_hbm)

    out = pl.pallas_call(
        functools.partial(_stage_e_body, n, grid_n),
        grid=(grid_n,),
        in_specs=[wspec, wspec, wspec, wspec, wspec, wspec,
                  pl.BlockSpec((blk, 64), lambda i: (i, 0)),
                  pl.BlockSpec(W7.shape, lambda i: (0, 0)),
                  pl.BlockSpec((1, 2), lambda i: (0, 0))],
        out_specs=pl.BlockSpec((1, 2), lambda i: (0, 0)),
        out_shape=jax.ShapeDtypeStruct((1, 2), jnp.float32),
        scratch_shapes=[pltpu.VMEM((8, 64), jnp.float32)],
    )(s20, s21, s22, s23, d0, d1, skip2, W7, b7.reshape(1, 2))
    return out
